# Initial kernel scaffold; baseline (speedup 1.0000x reference)
import jax
import jax.numpy as jnp
from jax.experimental import pallas as pl


def kernel(real, imag, L_real, L_imag, cheb_w0, cheb_b0, cheb_w1, cheb_b1, cheb_w2, cheb_b2, ca_w1, ca_w2, sa_w, conv_w, conv_b):
    raise NotImplementedError("write your pallas kernel here")


# speedup vs baseline: 1.6977x; 1.6977x over previous
import jax
import jax.numpy as jnp
from jax.experimental import pallas as pl
from jax.experimental.pallas import tpu as pltpu

_N = 2048
_K1 = 3
_F = 8
_C2 = 2 * _F


def _layer_body(xr_ref, xi_ref, w_ref, b_ref, lr_ref, li_ref, out_ref,
                accr_ref, acci_ref):
    i = pl.program_id(0)
    xrb = xr_ref[...].astype(jnp.bfloat16)
    xib = xi_ref[...].astype(jnp.bfloat16)
    lr16 = lr_ref[0]
    li16 = li_ref[0]
    wb = w_ref[0].astype(jnp.bfloat16)
    u_rr = jnp.dot(lr16, xrb, preferred_element_type=jnp.float32).astype(jnp.bfloat16)
    u_ii = jnp.dot(li16, xib, preferred_element_type=jnp.float32).astype(jnp.bfloat16)
    u_ir = jnp.dot(li16, xrb, preferred_element_type=jnp.float32).astype(jnp.bfloat16)
    u_ri = jnp.dot(lr16, xib, preferred_element_type=jnp.float32).astype(jnp.bfloat16)
    r = (jnp.dot(u_rr, wb, preferred_element_type=jnp.float32)
         - jnp.dot(u_ii, wb, preferred_element_type=jnp.float32))
    im = (jnp.dot(u_ir, wb, preferred_element_type=jnp.float32)
          + jnp.dot(u_ri, wb, preferred_element_type=jnp.float32))

    @pl.when(i == 0)
    def _init():
        accr_ref[...] = r
        acci_ref[...] = im

    @pl.when(i > 0)
    def _acc():
        accr_ref[...] += r
        acci_ref[...] += im

    @pl.when(i == _K1 - 1)
    def _finish():
        real = accr_ref[...] + b_ref[...]
        imag = acci_ref[...] + b_ref[...]
        mask = (real >= 0).astype(real.dtype)
        out_ref[...] = jnp.concatenate([mask * real, mask * imag], axis=-1)


def _mag_layer(xr, xi, w, b, lr16, li16):
    c = xr.shape[1]
    return pl.pallas_call(
        _layer_body,
        grid=(_K1,),
        in_specs=[
            pl.BlockSpec((_N, c), lambda i: (0, 0)),
            pl.BlockSpec((_N, c), lambda i: (0, 0)),
            pl.BlockSpec((1, c, _F), lambda i: (i, 0, 0)),
            pl.BlockSpec((1, _F), lambda i: (0, 0)),
            pl.BlockSpec((1, _N, _N), lambda i: (i, 0, 0)),
            pl.BlockSpec((1, _N, _N), lambda i: (i, 0, 0)),
        ],
        out_specs=pl.BlockSpec((_N, _C2), lambda i: (0, 0)),
        out_shape=jax.ShapeDtypeStruct((_N, _C2), jnp.float32),
        scratch_shapes=[
            pltpu.VMEM((_N, _F), jnp.float32),
            pltpu.VMEM((_N, _F), jnp.float32),
        ],
    )(xr, xi, w, b, lr16, li16)


def _tail_body(x_ref, caw1_ref, caw2_ref, saw_ref, cwt_ref, cb_ref, out_ref):
    x = x_ref[...]
    mean_c = jnp.mean(x, axis=0, keepdims=True)
    s = jnp.maximum(jnp.sum(mean_c * caw1_ref[...]), 0.0)
    ca = jax.nn.sigmoid(caw2_ref[...] * s)
    xs = x * ca
    m = jnp.mean(xs, axis=1, keepdims=True)
    mx = jnp.max(xs, axis=1, keepdims=True)
    zpad = jnp.zeros((3, 1), jnp.float32)
    mp = jnp.concatenate([zpad, m, zpad], axis=0)
    mxp = jnp.concatenate([zpad, mx, zpad], axis=0)
    acc = jnp.zeros((_N, 1), jnp.float32)
    for k in range(7):
        acc = acc + saw_ref[0, k] * mp[k:k + _N] + saw_ref[1, k] * mxp[k:k + _N]
    sp = jax.nn.sigmoid(acc)
    xsp = xs * sp
    logits = jnp.dot(xsp, cwt_ref[...],
                     preferred_element_type=jnp.float32) + cb_ref[...]
    zmax = jnp.max(logits, axis=1, keepdims=True)
    z = logits - zmax
    lse = jnp.log(jnp.sum(jnp.exp(z), axis=1, keepdims=True))
    out_ref[...] = z - lse


def _tail(x, ca_w1, ca_w2, sa_w, conv_w, conv_b):
    caw1 = ca_w1[:, :, 0]
    caw2 = ca_w2[:, 0, 0][None, :]
    saw = sa_w[0]
    cwt = conv_w[:, :, 0].T
    cb = conv_b[None, :]
    return pl.pallas_call(
        _tail_body,
        in_specs=[
            pl.BlockSpec(memory_space=pltpu.VMEM),
            pl.BlockSpec(memory_space=pltpu.VMEM),
            pl.BlockSpec(memory_space=pltpu.VMEM),
            pl.BlockSpec(memory_space=pltpu.SMEM),
            pl.BlockSpec(memory_space=pltpu.VMEM),
            pl.BlockSpec(memory_space=pltpu.VMEM),
        ],
        out_shape=jax.ShapeDtypeStruct((_N, 2), jnp.float32),
    )(x, caw1, caw2, saw, cwt, cb)


def kernel(real, imag, L_real, L_imag, cheb_w0, cheb_b0, cheb_w1, cheb_b1,
           cheb_w2, cheb_b2, ca_w1, ca_w2, sa_w, conv_w, conv_b):
    lr16 = L_real.astype(jnp.bfloat16)
    li16 = L_imag.astype(jnp.bfloat16)
    x = _mag_layer(real, imag, cheb_w0, cheb_b0, lr16, li16)
    x = _mag_layer(x[:, :_F], x[:, _F:], cheb_w1, cheb_b1, lr16, li16)
    x = _mag_layer(x[:, :_F], x[:, _F:], cheb_w2, cheb_b2, lr16, li16)
    y = _tail(x, ca_w1, ca_w2, sa_w, conv_w, conv_b)
    return jnp.transpose(y)[None, :, :]
